# baseline (speedup 1.0000x reference)
import functools

import jax
import jax.numpy as jnp
from jax.experimental import pallas as pl
from jax.experimental.pallas import tpu as pltpu

TOPK = 2
BM = 256
BI = 256


def _swiglu_kernel(meta_ref, xg_ref, w11_ref, w12_ref, w2_ref, wrow_ref,
                   out_ref):
    j = pl.program_id(0)
    u = pl.program_id(1)

    tile = meta_ref[1, u]
    start = meta_ref[2, u]
    end = meta_ref[3, u]
    first = meta_ref[4, u]

    xg = xg_ref[0]
    w11 = w11_ref[0]
    w12 = w12_ref[0]
    w2 = w2_ref[0]

    xgb = xg.astype(jnp.bfloat16)
    g = jnp.dot(xgb, w11.astype(jnp.bfloat16).T,
                preferred_element_type=jnp.float32)
    v = jnp.dot(xgb, w12.astype(jnp.bfloat16).T,
                preferred_element_type=jnp.float32)
    h = g * jax.nn.sigmoid(g) * v
    contrib = jnp.dot(h.astype(jnp.bfloat16), w2.astype(jnp.bfloat16).T,
                      preferred_element_type=jnp.float32)

    base = tile * BM
    rows = base + jax.lax.broadcasted_iota(jnp.int32, (BM, 1), 0)
    mask = (rows >= start) & (rows < end)
    contrib = jnp.where(mask, contrib * wrow_ref[0], 0.0)

    init = (first == 1) & (j == 0)

    @pl.when(init)
    def _():
        out_ref[pl.ds(base, BM), :] = contrib

    @pl.when(jnp.logical_not(init))
    def _():
        out_ref[pl.ds(base, BM), :] += contrib


@functools.partial(jax.jit, static_argnames=())
def kernel(x, gate_W, W11, W12, W2):
    B, S, H = x.shape
    E, I, _ = W11.shape
    xs = x.reshape(S, H)

    logits = jnp.dot(xs, gate_W.T)
    gate_scores = jax.nn.softmax(logits, axis=-1)
    top_v, top_i = jax.lax.top_k(gate_scores, TOPK)
    top_v = top_v / (jnp.sum(top_v, axis=-1, keepdims=True) + 1e-8)

    N = S * TOPK
    eid = top_i.reshape(N)
    wts = top_v.reshape(N)
    tok = jnp.repeat(jnp.arange(S, dtype=jnp.int32), TOPK)
    order = jnp.argsort(eid)
    eid_s = eid[order]
    tok_s = tok[order]
    w_s = wts[order]

    xg = jnp.take(xs, tok_s, axis=0)

    T = N // BM
    U = T + E - 1
    counts = jnp.sum(eid_s[None, :] == jnp.arange(E, dtype=eid_s.dtype)[:, None],
                     axis=1).astype(jnp.int32)
    ends = jnp.cumsum(counts)
    starts = ends - counts
    first_tile = starts // BM
    last_tile = jnp.maximum(first_tile, (ends - 1) // BM)
    nunits = jnp.where(counts > 0, last_tile - first_tile + 1, 0)
    u_off = jnp.cumsum(nunits)
    u_start = u_off - nunits

    uu = jnp.arange(U, dtype=jnp.int32)
    e_of_u = jnp.searchsorted(u_off, uu, side='right').astype(jnp.int32)
    valid = e_of_u < E
    e_clip = jnp.minimum(e_of_u, E - 1)
    tile_of_u = jnp.where(valid, first_tile[e_clip] + (uu - u_start[e_clip]),
                          T - 1).astype(jnp.int32)
    start_of_u = jnp.where(valid, starts[e_clip], 0).astype(jnp.int32)
    end_of_u = jnp.where(valid, ends[e_clip], 0).astype(jnp.int32)
    first_of_u = jnp.concatenate([
        jnp.ones((1,), jnp.bool_),
        tile_of_u[1:] != tile_of_u[:-1]]) & valid
    meta = jnp.stack([e_clip, tile_of_u, start_of_u, end_of_u,
                      first_of_u.astype(jnp.int32)])

    xg3 = xg.reshape(T, BM, H)
    wcol3 = w_s.reshape(T, BM, 1)

    NI = I // BI
    grid_spec = pltpu.PrefetchScalarGridSpec(
        num_scalar_prefetch=1,
        grid=(NI, U),
        in_specs=[
            pl.BlockSpec((1, BM, H), lambda j, u, m: (m[1, u], 0, 0)),
            pl.BlockSpec((1, BI, H), lambda j, u, m: (m[0, u], j, 0)),
            pl.BlockSpec((1, BI, H), lambda j, u, m: (m[0, u], j, 0)),
            pl.BlockSpec((1, H, BI), lambda j, u, m: (m[0, u], 0, j)),
            pl.BlockSpec((1, BM, 1), lambda j, u, m: (m[1, u], 0, 0)),
        ],
        out_specs=pl.BlockSpec((N, H), lambda j, u, m: (0, 0)),
    )
    out_sorted = pl.pallas_call(
        _swiglu_kernel,
        grid_spec=grid_spec,
        out_shape=jax.ShapeDtypeStruct((N, H), jnp.float32),
    )(meta, xg3, W11, W12, W2, wcol3)

    inv = jnp.argsort(order)
    out = (jnp.take(out_sorted, inv[0::TOPK], axis=0) +
           jnp.take(out_sorted, inv[1::TOPK], axis=0))
    return out.reshape(B, S, H)


# speedup vs baseline: 1.4611x; 1.4611x over previous
import functools

import jax
import jax.numpy as jnp
from jax.experimental import pallas as pl
from jax.experimental.pallas import tpu as pltpu

TOPK = 2
BM = 256
BI = 1408


def _swiglu_kernel(meta_ref, xg_ref, w11_ref, w12_ref, w2_ref, wrow_ref,
                   out_ref):
    j = pl.program_id(0)
    u = pl.program_id(1)

    tile = meta_ref[1, u]
    start = meta_ref[2, u]
    end = meta_ref[3, u]
    first = meta_ref[4, u]

    xg = xg_ref[0]
    w11 = w11_ref[0]
    w12 = w12_ref[0]
    w2 = w2_ref[0]

    g = jnp.dot(xg, w11.T, preferred_element_type=jnp.float32)
    v = jnp.dot(xg, w12.T, preferred_element_type=jnp.float32)
    h = g * jax.nn.sigmoid(g) * v
    contrib = jnp.dot(h, w2.T, preferred_element_type=jnp.float32)

    base = tile * BM
    rows = base + jax.lax.broadcasted_iota(jnp.int32, (BM, 1), 0)
    mask = (rows >= start) & (rows < end)
    contrib = jnp.where(mask, contrib * wrow_ref[0], 0.0)

    init = (first == 1) & (j == 0)

    @pl.when(init)
    def _():
        out_ref[pl.ds(base, BM), :] = contrib

    @pl.when(jnp.logical_not(init))
    def _():
        out_ref[pl.ds(base, BM), :] += contrib


@functools.partial(jax.jit, static_argnames=())
def kernel(x, gate_W, W11, W12, W2):
    B, S, H = x.shape
    E, I, _ = W11.shape
    xs = x.reshape(S, H)

    logits = jnp.dot(xs, gate_W.T)
    gate_scores = jax.nn.softmax(logits, axis=-1)
    top_v, top_i = jax.lax.top_k(gate_scores, TOPK)
    top_v = top_v / (jnp.sum(top_v, axis=-1, keepdims=True) + 1e-8)

    N = S * TOPK
    eid = top_i.reshape(N)
    wts = top_v.reshape(N)
    tok = jnp.repeat(jnp.arange(S, dtype=jnp.int32), TOPK)
    order = jnp.argsort(eid)
    eid_s = eid[order]
    tok_s = tok[order]
    w_s = wts[order]

    xg = jnp.take(xs, tok_s, axis=0)

    T = N // BM
    U = T + E - 1
    counts = jnp.sum(eid_s[None, :] == jnp.arange(E, dtype=eid_s.dtype)[:, None],
                     axis=1).astype(jnp.int32)
    ends = jnp.cumsum(counts)
    starts = ends - counts
    first_tile = starts // BM
    last_tile = jnp.maximum(first_tile, (ends - 1) // BM)
    nunits = jnp.where(counts > 0, last_tile - first_tile + 1, 0)
    u_off = jnp.cumsum(nunits)
    u_start = u_off - nunits

    uu = jnp.arange(U, dtype=jnp.int32)
    e_of_u = jnp.searchsorted(u_off, uu, side='right').astype(jnp.int32)
    valid = e_of_u < E
    e_clip = jnp.minimum(e_of_u, E - 1)
    tile_of_u = jnp.where(valid, first_tile[e_clip] + (uu - u_start[e_clip]),
                          T - 1).astype(jnp.int32)
    start_of_u = jnp.where(valid, starts[e_clip], 0).astype(jnp.int32)
    end_of_u = jnp.where(valid, ends[e_clip], 0).astype(jnp.int32)
    first_of_u = jnp.concatenate([
        jnp.ones((1,), jnp.bool_),
        tile_of_u[1:] != tile_of_u[:-1]]) & valid
    meta = jnp.stack([e_clip, tile_of_u, start_of_u, end_of_u,
                      first_of_u.astype(jnp.int32)])

    xg3 = xg.reshape(T, BM, H)
    wcol3 = w_s.reshape(T, BM, 1)

    NI = I // BI
    grid_spec = pltpu.PrefetchScalarGridSpec(
        num_scalar_prefetch=1,
        grid=(NI, U),
        in_specs=[
            pl.BlockSpec((1, BM, H), lambda j, u, m: (m[1, u], 0, 0)),
            pl.BlockSpec((1, BI, H), lambda j, u, m: (m[0, u], j, 0)),
            pl.BlockSpec((1, BI, H), lambda j, u, m: (m[0, u], j, 0)),
            pl.BlockSpec((1, H, BI), lambda j, u, m: (m[0, u], 0, j)),
            pl.BlockSpec((1, BM, 1), lambda j, u, m: (m[1, u], 0, 0)),
        ],
        out_specs=pl.BlockSpec((N, H), lambda j, u, m: (0, 0)),
    )
    out_sorted = pl.pallas_call(
        _swiglu_kernel,
        grid_spec=grid_spec,
        out_shape=jax.ShapeDtypeStruct((N, H), jnp.float32),
    )(meta, xg3, W11, W12, W2, wcol3)

    inv = jnp.argsort(order)
    out = (jnp.take(out_sorted, inv[0::TOPK], axis=0) +
           jnp.take(out_sorted, inv[1::TOPK], axis=0))
    return out.reshape(B, S, H)


# speedup vs baseline: 1.4852x; 1.0165x over previous
import functools

import jax
import jax.numpy as jnp
from jax.experimental import pallas as pl
from jax.experimental.pallas import tpu as pltpu

TOPK = 2
BM = 256
BI = 1408
UMAX = 128


def _routing_kernel(x_ref, gw_ref, packed_ref, meta_ref):
    S = x_ref.shape[0]
    E = gw_ref.shape[0]
    N = S * TOPK
    T = N // BM
    U = T + E - 1

    logits = jnp.dot(x_ref[...], gw_ref[...].T,
                     preferred_element_type=jnp.float32)
    mx = jnp.max(logits, axis=-1, keepdims=True)
    ex = jnp.exp(logits - mx)
    sc = ex / jnp.sum(ex, axis=-1, keepdims=True)

    lane = jax.lax.broadcasted_iota(jnp.int32, (S, E), 1)
    s0 = jnp.max(sc, axis=-1, keepdims=True)
    i0 = jnp.min(jnp.where(sc == s0, lane, E), axis=-1, keepdims=True)
    sc1 = jnp.where(lane == i0, -1.0, sc)
    s1 = jnp.max(sc1, axis=-1, keepdims=True)
    i1 = jnp.min(jnp.where(sc1 == s1, lane, E), axis=-1, keepdims=True)
    den = s0 + s1 + 1e-8
    w0 = s0 / den
    w1 = s1 / den

    oh = ((lane == i0) | (lane == i1)).astype(jnp.float32)
    rows_i = jax.lax.broadcasted_iota(jnp.int32, (S, S), 0)
    cols_i = jax.lax.broadcasted_iota(jnp.int32, (S, S), 1)
    tril = (cols_i < rows_i).astype(jnp.float32)
    excl = jnp.dot(tril, oh, preferred_element_type=jnp.float32)
    counts_row = jnp.sum(oh, axis=0, keepdims=True)
    counts_col = jax.lax.dot_general(
        oh, jnp.ones((S, 1), jnp.float32), (((0,), (0,)), ((), ())),
        preferred_element_type=jnp.float32)
    er = jax.lax.broadcasted_iota(jnp.int32, (E, E), 0)
    ec = jax.lax.broadcasted_iota(jnp.int32, (E, E), 1)
    seg_start = jnp.sum(jnp.where(er < ec, counts_col, 0.0), axis=0,
                        keepdims=True)
    pos = seg_start + excl
    r0 = jnp.sum(jnp.where(lane == i0, pos, 0.0), axis=-1, keepdims=True)
    r1 = jnp.sum(jnp.where(lane == i1, pos, 0.0), axis=-1, keepdims=True)

    out_lane = jax.lax.broadcasted_iota(jnp.int32, (S, 8), 1)
    packed = (jnp.where(out_lane == 0, r0, 0.0) +
              jnp.where(out_lane == 1, r1, 0.0) +
              jnp.where(out_lane == 2, w0, 0.0) +
              jnp.where(out_lane == 3, w1, 0.0))
    packed_ref[...] = packed

    ends_s = jnp.sum(jnp.where(ec <= er, counts_row, 0.0), axis=1,
                     keepdims=True).astype(jnp.int32)
    counts_s = counts_col.astype(jnp.int32)
    starts_s = ends_s - counts_s
    ft = starts_s // BM
    lt = jnp.maximum(ft, (ends_s - 1) // BM)
    nu = jnp.where(counts_s > 0, lt - ft + 1, 0)
    lincl = (ec <= er).astype(jnp.float32)
    uoff = jnp.dot(lincl, nu.astype(jnp.float32),
                   preferred_element_type=jnp.float32).astype(jnp.int32)
    ustart = uoff - nu

    u_iota = jax.lax.broadcasted_iota(jnp.int32, (1, UMAX), 1)
    e_of_u = jnp.sum((u_iota >= uoff).astype(jnp.int32), axis=0,
                     keepdims=True)
    sel = (jax.lax.broadcasted_iota(jnp.int32, (E, UMAX), 0) == e_of_u)

    def pick(v):
        return jnp.sum(jnp.where(sel, v, 0), axis=0, keepdims=True)

    ft_u = pick(ft)
    ustart_u = pick(ustart)
    start_u = pick(starts_s)
    end_u = pick(ends_s)
    valid = (e_of_u < E) & (u_iota < U)
    tile_u = jnp.where(valid, ft_u + (u_iota - ustart_u), T - 1)
    first_u = valid & jnp.logical_not((u_iota == ustart_u) &
                                      (start_u % BM != 0))
    zero = jnp.zeros((1, UMAX), jnp.int32)
    meta_ref[...] = jnp.concatenate([
        jnp.where(valid, e_of_u, E - 1),
        tile_u,
        jnp.where(valid, start_u, 0),
        jnp.where(valid, end_u, 0),
        first_u.astype(jnp.int32),
        zero, zero, zero], axis=0)


def _swiglu_kernel(meta_ref, xg_ref, w11_ref, w12_ref, w2_ref, out_ref):
    j = pl.program_id(0)
    u = pl.program_id(1)

    tile = meta_ref[1, u]
    start = meta_ref[2, u]
    end = meta_ref[3, u]
    first = meta_ref[4, u]

    xg = xg_ref[0]
    w11 = w11_ref[0]
    w12 = w12_ref[0]
    w2 = w2_ref[0]

    g = jnp.dot(xg, w11.T, preferred_element_type=jnp.float32)
    v = jnp.dot(xg, w12.T, preferred_element_type=jnp.float32)
    h = g * jax.nn.sigmoid(g) * v
    contrib = jnp.dot(h, w2.T, preferred_element_type=jnp.float32)

    base = tile * BM
    rows = base + jax.lax.broadcasted_iota(jnp.int32, (BM, 1), 0)
    mask = (rows >= start) & (rows < end)
    contrib = jnp.where(mask, contrib, 0.0)

    init = (first == 1) & (j == 0)

    @pl.when(init)
    def _():
        out_ref[pl.ds(base, BM), :] = contrib

    @pl.when(jnp.logical_not(init))
    def _():
        out_ref[pl.ds(base, BM), :] += contrib


@functools.partial(jax.jit, static_argnames=())
def kernel(x, gate_W, W11, W12, W2):
    B, S, H = x.shape
    E, I, _ = W11.shape
    xs = x.reshape(S, H)
    N = S * TOPK
    T = N // BM
    U = T + E - 1

    packed, meta8 = pl.pallas_call(
        _routing_kernel,
        out_shape=(jax.ShapeDtypeStruct((S, 8), jnp.float32),
                   jax.ShapeDtypeStruct((8, UMAX), jnp.int32)),
    )(xs, gate_W)

    r0 = packed[:, 0].astype(jnp.int32)
    r1 = packed[:, 1].astype(jnp.int32)
    w0 = packed[:, 2:3]
    w1 = packed[:, 3:4]
    meta = meta8[:5, :U]

    rank_flat = jnp.stack([r0, r1], axis=1).reshape(N)
    tok = jnp.repeat(jnp.arange(S, dtype=jnp.int32), TOPK)
    tok_s = jnp.zeros((N,), jnp.int32).at[rank_flat].set(
        tok, unique_indices=True)
    xg = jnp.take(xs, tok_s, axis=0)
    xg3 = xg.reshape(T, BM, H)

    NI = I // BI
    grid_spec = pltpu.PrefetchScalarGridSpec(
        num_scalar_prefetch=1,
        grid=(NI, U),
        in_specs=[
            pl.BlockSpec((1, BM, H), lambda j, u, m: (m[1, u], 0, 0)),
            pl.BlockSpec((1, BI, H), lambda j, u, m: (m[0, u], j, 0)),
            pl.BlockSpec((1, BI, H), lambda j, u, m: (m[0, u], j, 0)),
            pl.BlockSpec((1, H, BI), lambda j, u, m: (m[0, u], 0, j)),
        ],
        out_specs=pl.BlockSpec((N, H), lambda j, u, m: (0, 0)),
    )
    out_sorted = pl.pallas_call(
        _swiglu_kernel,
        grid_spec=grid_spec,
        out_shape=jax.ShapeDtypeStruct((N, H), jnp.float32),
    )(meta, xg3, W11, W12, W2)

    out = (w0 * jnp.take(out_sorted, r0, axis=0) +
           w1 * jnp.take(out_sorted, r1, axis=0))
    return out.reshape(B, S, H)


# speedup vs baseline: 1.5771x; 1.0619x over previous
import functools

import jax
import jax.numpy as jnp
from jax.experimental import pallas as pl
from jax.experimental.pallas import tpu as pltpu

TOPK = 2
BM = 256
BI = 1408
UMAX = 128


def _routing_kernel(x_ref, gw_ref, packed_ref, meta_ref):
    S = x_ref.shape[0]
    E = gw_ref.shape[0]
    N = S * TOPK
    T = N // BM
    U = T + E - 1

    logits = jnp.dot(x_ref[...], gw_ref[...].T,
                     preferred_element_type=jnp.float32)
    mx = jnp.max(logits, axis=-1, keepdims=True)
    ex = jnp.exp(logits - mx)
    sc = ex / jnp.sum(ex, axis=-1, keepdims=True)

    lane = jax.lax.broadcasted_iota(jnp.int32, (S, E), 1)
    s0 = jnp.max(sc, axis=-1, keepdims=True)
    i0 = jnp.min(jnp.where(sc == s0, lane, E), axis=-1, keepdims=True)
    sc1 = jnp.where(lane == i0, -1.0, sc)
    s1 = jnp.max(sc1, axis=-1, keepdims=True)
    i1 = jnp.min(jnp.where(sc1 == s1, lane, E), axis=-1, keepdims=True)
    den = s0 + s1 + 1e-8
    w0 = s0 / den
    w1 = s1 / den

    oh = ((lane == i0) | (lane == i1)).astype(jnp.float32)
    rows_i = jax.lax.broadcasted_iota(jnp.int32, (S, S), 0)
    cols_i = jax.lax.broadcasted_iota(jnp.int32, (S, S), 1)
    tril = (cols_i < rows_i).astype(jnp.float32)
    excl = jnp.dot(tril, oh, preferred_element_type=jnp.float32)
    counts_row = jnp.sum(oh, axis=0, keepdims=True)
    counts_col = jax.lax.dot_general(
        oh, jnp.ones((S, 1), jnp.float32), (((0,), (0,)), ((), ())),
        preferred_element_type=jnp.float32)
    er = jax.lax.broadcasted_iota(jnp.int32, (E, E), 0)
    ec = jax.lax.broadcasted_iota(jnp.int32, (E, E), 1)
    seg_start = jnp.sum(jnp.where(er < ec, counts_col, 0.0), axis=0,
                        keepdims=True)
    pos = seg_start + excl
    r0 = jnp.sum(jnp.where(lane == i0, pos, 0.0), axis=-1, keepdims=True)
    r1 = jnp.sum(jnp.where(lane == i1, pos, 0.0), axis=-1, keepdims=True)

    out_lane = jax.lax.broadcasted_iota(jnp.int32, (S, 8), 1)
    packed = (jnp.where(out_lane == 0, r0, 0.0) +
              jnp.where(out_lane == 1, r1, 0.0) +
              jnp.where(out_lane == 2, w0, 0.0) +
              jnp.where(out_lane == 3, w1, 0.0))
    packed_ref[...] = packed

    ends_s = jnp.sum(jnp.where(ec <= er, counts_row, 0.0), axis=1,
                     keepdims=True).astype(jnp.int32)
    counts_s = counts_col.astype(jnp.int32)
    starts_s = ends_s - counts_s
    ft = starts_s // BM
    lt = jnp.maximum(ft, (ends_s - 1) // BM)
    nu = jnp.where(counts_s > 0, lt - ft + 1, 0)
    lincl = (ec <= er).astype(jnp.float32)
    uoff = jnp.dot(lincl, nu.astype(jnp.float32),
                   preferred_element_type=jnp.float32).astype(jnp.int32)
    ustart = uoff - nu

    u_iota = jax.lax.broadcasted_iota(jnp.int32, (1, UMAX), 1)
    e_of_u = jnp.sum((u_iota >= uoff).astype(jnp.int32), axis=0,
                     keepdims=True)
    sel = (jax.lax.broadcasted_iota(jnp.int32, (E, UMAX), 0) == e_of_u)

    def pick(v):
        return jnp.sum(jnp.where(sel, v, 0), axis=0, keepdims=True)

    ft_u = pick(ft)
    ustart_u = pick(ustart)
    start_u = pick(starts_s)
    end_u = pick(ends_s)
    valid = (e_of_u < E) & (u_iota < U)
    tile_u = jnp.where(valid, ft_u + (u_iota - ustart_u), T - 1)
    first_u = valid & jnp.logical_not((u_iota == ustart_u) &
                                      (start_u % BM != 0))
    zero = jnp.zeros((1, UMAX), jnp.int32)
    meta_ref[...] = jnp.concatenate([
        jnp.where(valid, e_of_u, E - 1),
        tile_u,
        jnp.where(valid, start_u, 0),
        jnp.where(valid, end_u, 0),
        first_u.astype(jnp.int32),
        zero, zero, zero], axis=0)


def _swiglu_kernel(meta_ref, xg_ref, w11_ref, w12_ref, w2_ref, out_ref):
    j = pl.program_id(0)
    u = pl.program_id(1)

    tile = meta_ref[1, u]
    start = meta_ref[2, u]
    end = meta_ref[3, u]
    first = meta_ref[4, u]

    xg = xg_ref[0]
    w11 = w11_ref[0]
    w12 = w12_ref[0]
    w2 = w2_ref[0]

    g = jnp.dot(xg, w11.T, preferred_element_type=jnp.float32)
    v = jnp.dot(xg, w12.T, preferred_element_type=jnp.float32)
    h = g * jax.nn.sigmoid(g) * v
    contrib = jnp.dot(h, w2.T, preferred_element_type=jnp.float32)

    base = tile * BM
    rows = base + jax.lax.broadcasted_iota(jnp.int32, (BM, 1), 0)
    mask = (rows >= start) & (rows < end)
    contrib = jnp.where(mask, contrib, 0.0)

    init = (first == 1) & (j == 0)

    contrib16 = contrib.astype(jnp.bfloat16)

    @pl.when(init)
    def _():
        out_ref[pl.ds(base, BM), :] = contrib16

    @pl.when(jnp.logical_not(init))
    def _():
        out_ref[pl.ds(base, BM), :] += contrib16


@functools.partial(jax.jit, static_argnames=())
def kernel(x, gate_W, W11, W12, W2):
    B, S, H = x.shape
    E, I, _ = W11.shape
    xs = x.reshape(S, H)
    N = S * TOPK
    T = N // BM
    U = T + E - 1

    packed, meta8 = pl.pallas_call(
        _routing_kernel,
        out_shape=(jax.ShapeDtypeStruct((S, 8), jnp.float32),
                   jax.ShapeDtypeStruct((8, UMAX), jnp.int32)),
    )(xs, gate_W)

    r0 = packed[:, 0].astype(jnp.int32)
    r1 = packed[:, 1].astype(jnp.int32)
    w0 = packed[:, 2:3]
    w1 = packed[:, 3:4]
    meta = meta8[:5, :U]

    rank_flat = jnp.stack([r0, r1], axis=1).reshape(N)
    tok = jnp.repeat(jnp.arange(S, dtype=jnp.int32), TOPK)
    tok_s = jnp.zeros((N,), jnp.int32).at[rank_flat].set(
        tok, unique_indices=True)
    xg = jnp.take(xs.astype(jnp.bfloat16), tok_s, axis=0)
    xg3 = xg.reshape(T, BM, H)

    NI = I // BI
    grid_spec = pltpu.PrefetchScalarGridSpec(
        num_scalar_prefetch=1,
        grid=(NI, U),
        in_specs=[
            pl.BlockSpec((1, BM, H), lambda j, u, m: (m[1, u], 0, 0)),
            pl.BlockSpec((1, BI, H), lambda j, u, m: (m[0, u], j, 0)),
            pl.BlockSpec((1, BI, H), lambda j, u, m: (m[0, u], j, 0)),
            pl.BlockSpec((1, H, BI), lambda j, u, m: (m[0, u], 0, j)),
        ],
        out_specs=pl.BlockSpec((N, H), lambda j, u, m: (0, 0)),
    )
    out_sorted = pl.pallas_call(
        _swiglu_kernel,
        grid_spec=grid_spec,
        out_shape=jax.ShapeDtypeStruct((N, H), jnp.bfloat16),
    )(meta, xg3, W11, W12, W2)

    out = (w0 * jnp.take(out_sorted, r0, axis=0).astype(jnp.float32) +
           w1 * jnp.take(out_sorted, r1, axis=0).astype(jnp.float32))
    return out.reshape(B, S, H)


# speedup vs baseline: 1.5850x; 1.0050x over previous
import functools

import jax
import jax.numpy as jnp
from jax.experimental import pallas as pl
from jax.experimental.pallas import tpu as pltpu

TOPK = 2
BM = 256
BI = 1408
UMAX = 128


def _routing_kernel(x_ref, gw_ref, packed_ref, meta_ref, xb_ref):
    S = x_ref.shape[0]
    E = gw_ref.shape[0]
    N = S * TOPK
    T = N // BM
    U = T + E - 1

    xv = x_ref[...]
    xb_ref[...] = xv.astype(jnp.bfloat16)
    logits = jnp.dot(xv, gw_ref[...].T,
                     preferred_element_type=jnp.float32)
    mx = jnp.max(logits, axis=-1, keepdims=True)
    ex = jnp.exp(logits - mx)
    sc = ex / jnp.sum(ex, axis=-1, keepdims=True)

    lane = jax.lax.broadcasted_iota(jnp.int32, (S, E), 1)
    s0 = jnp.max(sc, axis=-1, keepdims=True)
    i0 = jnp.min(jnp.where(sc == s0, lane, E), axis=-1, keepdims=True)
    sc1 = jnp.where(lane == i0, -1.0, sc)
    s1 = jnp.max(sc1, axis=-1, keepdims=True)
    i1 = jnp.min(jnp.where(sc1 == s1, lane, E), axis=-1, keepdims=True)
    den = s0 + s1 + 1e-8
    w0 = s0 / den
    w1 = s1 / den

    oh = ((lane == i0) | (lane == i1)).astype(jnp.float32)
    rows_i = jax.lax.broadcasted_iota(jnp.int32, (S, S), 0)
    cols_i = jax.lax.broadcasted_iota(jnp.int32, (S, S), 1)
    tril = (cols_i < rows_i).astype(jnp.float32)
    excl = jnp.dot(tril, oh, preferred_element_type=jnp.float32)
    counts_row = jnp.sum(oh, axis=0, keepdims=True)
    counts_col = jax.lax.dot_general(
        oh, jnp.ones((S, 1), jnp.float32), (((0,), (0,)), ((), ())),
        preferred_element_type=jnp.float32)
    er = jax.lax.broadcasted_iota(jnp.int32, (E, E), 0)
    ec = jax.lax.broadcasted_iota(jnp.int32, (E, E), 1)
    seg_start = jnp.sum(jnp.where(er < ec, counts_col, 0.0), axis=0,
                        keepdims=True)
    pos = seg_start + excl
    r0 = jnp.sum(jnp.where(lane == i0, pos, 0.0), axis=-1, keepdims=True)
    r1 = jnp.sum(jnp.where(lane == i1, pos, 0.0), axis=-1, keepdims=True)

    out_lane = jax.lax.broadcasted_iota(jnp.int32, (S, 8), 1)
    packed = (jnp.where(out_lane == 0, r0, 0.0) +
              jnp.where(out_lane == 1, r1, 0.0) +
              jnp.where(out_lane == 2, w0, 0.0) +
              jnp.where(out_lane == 3, w1, 0.0))
    packed_ref[...] = packed

    ends_s = jnp.sum(jnp.where(ec <= er, counts_row, 0.0), axis=1,
                     keepdims=True).astype(jnp.int32)
    counts_s = counts_col.astype(jnp.int32)
    starts_s = ends_s - counts_s
    ft = starts_s // BM
    lt = jnp.maximum(ft, (ends_s - 1) // BM)
    nu = jnp.where(counts_s > 0, lt - ft + 1, 0)
    lincl = (ec <= er).astype(jnp.float32)
    uoff = jnp.dot(lincl, nu.astype(jnp.float32),
                   preferred_element_type=jnp.float32).astype(jnp.int32)
    ustart = uoff - nu

    u_iota = jax.lax.broadcasted_iota(jnp.int32, (1, UMAX), 1)
    e_of_u = jnp.sum((u_iota >= uoff).astype(jnp.int32), axis=0,
                     keepdims=True)
    sel = (jax.lax.broadcasted_iota(jnp.int32, (E, UMAX), 0) == e_of_u)

    def pick(v):
        return jnp.sum(jnp.where(sel, v, 0), axis=0, keepdims=True)

    ft_u = pick(ft)
    ustart_u = pick(ustart)
    start_u = pick(starts_s)
    end_u = pick(ends_s)
    valid = (e_of_u < E) & (u_iota < U)
    tile_u = jnp.where(valid, ft_u + (u_iota - ustart_u), T - 1)
    first_u = valid & jnp.logical_not((u_iota == ustart_u) &
                                      (start_u % BM != 0))
    zero = jnp.zeros((1, UMAX), jnp.int32)
    meta_ref[...] = jnp.concatenate([
        jnp.where(valid, e_of_u, E - 1),
        tile_u,
        jnp.where(valid, start_u, 0),
        jnp.where(valid, end_u, 0),
        first_u.astype(jnp.int32),
        zero, zero, zero], axis=0)


def _swiglu_kernel(meta_ref, xg_ref, w11_ref, w12_ref, w2_ref, out_ref):
    j = pl.program_id(0)
    u = pl.program_id(1)

    tile = meta_ref[1, u]
    start = meta_ref[2, u]
    end = meta_ref[3, u]
    first = meta_ref[4, u]

    xg = xg_ref[0]
    w11 = w11_ref[0]
    w12 = w12_ref[0]
    w2 = w2_ref[0]

    g = jnp.dot(xg, w11.T, preferred_element_type=jnp.float32)
    v = jnp.dot(xg, w12.T, preferred_element_type=jnp.float32)
    h = g * jax.nn.sigmoid(g) * v
    contrib = jnp.dot(h, w2.T, preferred_element_type=jnp.float32)

    base = tile * BM
    rows = base + jax.lax.broadcasted_iota(jnp.int32, (BM, 1), 0)
    mask = (rows >= start) & (rows < end)
    contrib = jnp.where(mask, contrib, 0.0)

    init = (first == 1) & (j == 0)

    contrib16 = contrib.astype(jnp.bfloat16)

    @pl.when(init)
    def _():
        out_ref[pl.ds(base, BM), :] = contrib16

    @pl.when(jnp.logical_not(init))
    def _():
        out_ref[pl.ds(base, BM), :] += contrib16


@functools.partial(jax.jit, static_argnames=())
def kernel(x, gate_W, W11, W12, W2):
    B, S, H = x.shape
    E, I, _ = W11.shape
    xs = x.reshape(S, H)
    N = S * TOPK
    T = N // BM
    U = T + E - 1

    packed, meta8, xb = pl.pallas_call(
        _routing_kernel,
        out_shape=(jax.ShapeDtypeStruct((S, 8), jnp.float32),
                   jax.ShapeDtypeStruct((8, UMAX), jnp.int32),
                   jax.ShapeDtypeStruct((S, H), jnp.bfloat16)),
    )(xs, gate_W)

    r0 = packed[:, 0].astype(jnp.int32)
    r1 = packed[:, 1].astype(jnp.int32)
    w0 = packed[:, 2:3]
    w1 = packed[:, 3:4]
    meta = meta8[:5, :U]

    rank_flat = jnp.stack([r0, r1], axis=1).reshape(N)
    tok = jnp.repeat(jnp.arange(S, dtype=jnp.int32), TOPK)
    tok_s = jnp.zeros((N,), jnp.int32).at[rank_flat].set(
        tok, unique_indices=True)
    xg = jnp.take(xb, tok_s, axis=0)
    xg3 = xg.reshape(T, BM, H)

    NI = I // BI
    grid_spec = pltpu.PrefetchScalarGridSpec(
        num_scalar_prefetch=1,
        grid=(NI, U),
        in_specs=[
            pl.BlockSpec((1, BM, H), lambda j, u, m: (m[1, u], 0, 0)),
            pl.BlockSpec((1, BI, H), lambda j, u, m: (m[0, u], j, 0)),
            pl.BlockSpec((1, BI, H), lambda j, u, m: (m[0, u], j, 0)),
            pl.BlockSpec((1, H, BI), lambda j, u, m: (m[0, u], 0, j)),
        ],
        out_specs=pl.BlockSpec((N, H), lambda j, u, m: (0, 0)),
    )
    out_sorted = pl.pallas_call(
        _swiglu_kernel,
        grid_spec=grid_spec,
        out_shape=jax.ShapeDtypeStruct((N, H), jnp.bfloat16),
    )(meta, xg3, W11, W12, W2)

    out = (w0 * jnp.take(out_sorted, r0, axis=0).astype(jnp.float32) +
           w1 * jnp.take(out_sorted, r1, axis=0).astype(jnp.float32))
    return out.reshape(B, S, H)
